# baseline (speedup 1.0000x reference)
import functools

import jax
import jax.numpy as jnp
from jax import lax
from jax.experimental import pallas as pl
from jax.experimental.pallas import tpu as pltpu
from jax.experimental.pallas import tpu_sc as plsc

B_ROWS = 16384
SEQ = 200
D = 64
TOTAL = B_ROWS * SEQ
IDX_MINOR = 128
ROWS = 21504
NUM_WORKERS = 32
ROWS_PER_W = ROWS // NUM_WORKERS
S = 2
STEPS = ROWS_PER_W // S
NBUF = 4


def _make_gather():
    mesh = plsc.VectorSubcoreMesh(core_axis_name="c", subcore_axis_name="s")

    @functools.partial(
        pl.kernel,
        mesh=mesh,
        out_type=jax.ShapeDtypeStruct((ROWS, IDX_MINOR, D), jnp.float32),
        scratch_types=[
            pltpu.VMEM((NBUF, S, IDX_MINOR), jnp.int32),
            pltpu.VMEM((NBUF, S, IDX_MINOR, D), jnp.float32),
            pltpu.SemaphoreType.DMA((NBUF,)),
            pltpu.SemaphoreType.DMA((NBUF,)),
            pltpu.SemaphoreType.DMA((NBUF,)),
        ],
        compiler_params=pltpu.CompilerParams(use_tc_tiling_on_sc=False),
    )
    def gather_kernel(idx_hbm, table_hbm, out_hbm, idx_v, rows_v,
                      sem_i, sem_g, sem_o):
        wid = lax.axis_index("s") * 2 + lax.axis_index("c")
        base = wid * ROWS_PER_W

        def idx_cp(step, b):
            return pltpu.make_async_copy(
                idx_hbm.at[pl.ds(base + step * S, S)], idx_v.at[b], sem_i.at[b])

        def gather_cp(b, j):
            return pltpu.make_async_copy(
                table_hbm.at[idx_v.at[b].at[j]], rows_v.at[b].at[j],
                sem_g.at[b])

        def store_cp(step, b):
            return pltpu.make_async_copy(
                rows_v.at[b], out_hbm.at[pl.ds(base + step * S, S)], sem_o.at[b])

        for b in range(NBUF):
            idx_cp(b, b).start()

        def body(i, carry):
            for b in range(NBUF):
                s = NBUF * i + b
                idx_cp(s, b).wait()

                @pl.when(i > 0)
                def _():
                    store_cp(s - NBUF, b).wait()

                for j in range(S):
                    gather_cp(b, j).start()
            for b in range(NBUF):
                s = NBUF * i + b
                for j in range(S):
                    gather_cp(b, j).wait()
                store_cp(s, b).start()

                @pl.when(s + NBUF < STEPS)
                def _():
                    idx_cp(s + NBUF, b).start()

            return carry

        lax.fori_loop(0, STEPS // NBUF, body, 0)

        for b in range(NBUF):
            store_cp(STEPS - NBUF + b, b).wait()

    return gather_kernel


_gather = _make_gather()


def kernel(idxes, pe):
    idx2 = idxes.reshape(TOTAL // IDX_MINOR, IDX_MINOR)
    out_sc = _gather(idx2[:ROWS], pe)
    out_tc = jnp.take(pe, idx2[ROWS:], axis=0)
    out = jnp.concatenate([out_sc, out_tc], axis=0)
    return out.reshape(B_ROWS, SEQ, D)
